# baseline (speedup 1.0000x reference)
import functools

import jax
import jax.numpy as jnp
from jax import lax
from jax.experimental import pallas as pl
from jax.experimental.pallas import tpu as pltpu
from jax.experimental.pallas import tpu_sc as plsc

_DIM = 768
_NE = 8
_TOKENS = 32768
_BLK = 4096

_NW = 32
_CHUNK = _TOKENS // _NW
_L = 16
_GROUPS = _CHUNK // _L


def _logits_body(x_ref, w_ref, b_ref, out_ref):
    out_ref[...] = jnp.dot(x_ref[...], w_ref[...]) + b_ref[...]


def _tc_logits(x, w, b2):
    return pl.pallas_call(
        _logits_body,
        grid=(_TOKENS // _BLK,),
        in_specs=[
            pl.BlockSpec((_BLK, _DIM), lambda i: (i, 0)),
            pl.BlockSpec((_DIM, _NE), lambda i: (0, 0)),
            pl.BlockSpec((1, _NE), lambda i: (0, 0)),
        ],
        out_specs=pl.BlockSpec((_BLK, _NE), lambda i: (i, 0)),
        out_shape=jax.ShapeDtypeStruct((_TOKENS, _NE), jnp.float32),
        compiler_params=pltpu.CompilerParams(
            dimension_semantics=("arbitrary",),
        ),
    )(x, w, b2)


@functools.partial(
    pl.kernel,
    mesh=plsc.VectorSubcoreMesh(core_axis_name="c", subcore_axis_name="s"),
    out_type=[
        jax.ShapeDtypeStruct((_TOKENS * _NE,), jnp.float32),
        jax.ShapeDtypeStruct((_TOKENS,), jnp.int32),
    ],
    scratch_types=[
        pltpu.VMEM((_CHUNK * _NE,), jnp.float32),
        pltpu.VMEM((_CHUNK * _NE,), jnp.float32),
        pltpu.VMEM((_CHUNK,), jnp.int32),
    ],
    compiler_params=pltpu.CompilerParams(needs_layout_passes=False),
)
def _sc_route(logits_hbm, router_hbm, idx_hbm, lg_v, rt_v, ix_v):
    wid = lax.axis_index("s") * 2 + lax.axis_index("c")
    base = wid * _CHUNK
    pltpu.sync_copy(logits_hbm.at[pl.ds(base * _NE, _CHUNK * _NE)], lg_v)

    lanes = lax.iota(jnp.int32, _L)

    def _group(g, carry):
        rows8 = (g * _L + lanes) * _NE
        best = plsc.load_gather(lg_v, [rows8])
        bidx = jnp.zeros((_L,), jnp.int32)
        for e in range(1, _NE):
            v = plsc.load_gather(lg_v, [rows8 + e])
            m = v > best
            best = jnp.where(m, v, best)
            bidx = jnp.where(m, e, bidx)
        ix_v[pl.ds(g * _L, _L)] = bidx
        for e in range(_NE):
            plsc.store_scatter(
                rt_v,
                [rows8 + e],
                jnp.where(bidx == e, 1.0, 0.0).astype(jnp.float32),
            )
        return carry

    lax.fori_loop(0, _GROUPS, _group, 0)

    pltpu.sync_copy(rt_v, router_hbm.at[pl.ds(base * _NE, _CHUNK * _NE)])
    pltpu.sync_copy(ix_v, idx_hbm.at[pl.ds(base, _CHUNK)])


def kernel(x, W, b):
    b2 = b.reshape(1, _NE)
    logits = _tc_logits(x, W, b2)
    router_flat, idx = _sc_route(logits.reshape(_TOKENS * _NE))
    return (router_flat.reshape(_TOKENS, _NE), idx.reshape(_TOKENS, 1))


# speedup vs baseline: 1.0009x; 1.0009x over previous
import functools

import jax
import jax.numpy as jnp
from jax import lax
from jax.experimental import pallas as pl
from jax.experimental.pallas import tpu as pltpu
from jax.experimental.pallas import tpu_sc as plsc

_DIM = 768
_NE = 8
_TOKENS = 32768
_BLK = 4096

_NW = 32
_CHUNK = _TOKENS // _NW
_L = 16
_GROUPS = _CHUNK // _L


def _logits_body(x_ref, w_ref, b_ref, out_ref):
    out_ref[...] = jnp.dot(x_ref[...], w_ref[...]) + b_ref[...]


def _tc_logits(x, w, b2):
    return pl.pallas_call(
        _logits_body,
        grid=(_TOKENS // _BLK,),
        in_specs=[
            pl.BlockSpec((_BLK, _DIM), lambda i: (i, 0)),
            pl.BlockSpec((_DIM, _NE), lambda i: (0, 0)),
            pl.BlockSpec((1, _NE), lambda i: (0, 0)),
        ],
        out_specs=pl.BlockSpec((_BLK, _NE), lambda i: (i, 0)),
        out_shape=jax.ShapeDtypeStruct((_TOKENS, _NE), jnp.float32),
        compiler_params=pltpu.CompilerParams(
            dimension_semantics=("arbitrary",),
        ),
    )(x, w, b2)


@functools.partial(
    pl.kernel,
    mesh=plsc.VectorSubcoreMesh(core_axis_name="c", subcore_axis_name="s"),
    out_type=[
        jax.ShapeDtypeStruct((_TOKENS * _NE,), jnp.float32),
        jax.ShapeDtypeStruct((_TOKENS,), jnp.int32),
    ],
    scratch_types=[
        pltpu.VMEM((_CHUNK * _NE,), jnp.float32),
        pltpu.VMEM((_CHUNK * _NE,), jnp.float32),
        pltpu.VMEM((_CHUNK,), jnp.int32),
    ],
    compiler_params=pltpu.CompilerParams(needs_layout_passes=False, skip_device_barrier=True),
)
def _sc_route(logits_hbm, router_hbm, idx_hbm, lg_v, rt_v, ix_v):
    wid = lax.axis_index("s") * 2 + lax.axis_index("c")
    base = wid * _CHUNK
    pltpu.sync_copy(logits_hbm.at[pl.ds(base * _NE, _CHUNK * _NE)], lg_v)

    lanes = lax.iota(jnp.int32, _L)

    def _group(g, carry):
        rows8 = (g * _L + lanes) * _NE
        best = plsc.load_gather(lg_v, [rows8])
        bidx = jnp.zeros((_L,), jnp.int32)
        for e in range(1, _NE):
            v = plsc.load_gather(lg_v, [rows8 + e])
            m = v > best
            best = jnp.where(m, v, best)
            bidx = jnp.where(m, e, bidx)
        ix_v[pl.ds(g * _L, _L)] = bidx
        for e in range(_NE):
            plsc.store_scatter(
                rt_v,
                [rows8 + e],
                jnp.where(bidx == e, 1.0, 0.0).astype(jnp.float32),
            )
        return carry

    lax.fori_loop(0, _GROUPS, _group, 0)

    pltpu.sync_copy(rt_v, router_hbm.at[pl.ds(base * _NE, _CHUNK * _NE)])
    pltpu.sync_copy(ix_v, idx_hbm.at[pl.ds(base, _CHUNK)])


def kernel(x, W, b):
    b2 = b.reshape(1, _NE)
    logits = _tc_logits(x, W, b2)
    router_flat, idx = _sc_route(logits.reshape(_TOKENS * _NE))
    return (router_flat.reshape(_TOKENS, _NE), idx.reshape(_TOKENS, 1))


# speedup vs baseline: 1.4345x; 1.4333x over previous
import jax
import jax.numpy as jnp
from jax.experimental import pallas as pl
from jax.experimental.pallas import tpu as pltpu

_DIM = 768
_NE = 8
_TOKENS = 32768
_BLK = 4096


def _router_body(x_ref, w_ref, b_ref, router_ref, idx_ref):
    logits = jnp.dot(x_ref[...], w_ref[...]) + b_ref[...]
    mx = jnp.max(logits, axis=1, keepdims=True)
    ids = jax.lax.broadcasted_iota(jnp.int32, (_BLK, _NE), 1)
    idx = jnp.min(jnp.where(logits == mx, ids, _NE), axis=1, keepdims=True)
    router_ref[...] = (ids == idx).astype(jnp.float32)
    idx_ref[...] = idx.reshape(_BLK)


def kernel(x, W, b):
    b2 = b.reshape(1, _NE)
    grid = (_TOKENS // _BLK,)
    router, idx = pl.pallas_call(
        _router_body,
        grid=grid,
        in_specs=[
            pl.BlockSpec((_BLK, _DIM), lambda i: (i, 0)),
            pl.BlockSpec((_DIM, _NE), lambda i: (0, 0)),
            pl.BlockSpec((1, _NE), lambda i: (0, 0)),
        ],
        out_specs=[
            pl.BlockSpec((_BLK, _NE), lambda i: (i, 0)),
            pl.BlockSpec((_BLK,), lambda i: (i,)),
        ],
        out_shape=[
            jax.ShapeDtypeStruct((_TOKENS, _NE), jnp.float32),
            jax.ShapeDtypeStruct((_TOKENS,), jnp.int32),
        ],
        compiler_params=pltpu.CompilerParams(
            dimension_semantics=("arbitrary",),
        ),
    )(x, W, b2)
    return (router, idx.reshape(_TOKENS, 1))


# speedup vs baseline: 1.4380x; 1.0025x over previous
import jax
import jax.numpy as jnp
from jax.experimental import pallas as pl
from jax.experimental.pallas import tpu as pltpu

_DIM = 768
_NE = 8
_TOKENS = 32768
_BLK = 4096


def _router_body(x_ref, w_ref, b_ref, router_ref, idx_ref):
    logits = jnp.dot(x_ref[...], w_ref[...]) + b_ref[...]
    mx = jnp.max(logits, axis=1, keepdims=True)
    ids = jax.lax.broadcasted_iota(jnp.int32, (_BLK, _NE), 1)
    idx = jnp.min(jnp.where(logits == mx, ids, _NE), axis=1, keepdims=True)
    router_ref[...] = (ids == idx).astype(jnp.float32)
    idx_ref[...] = jnp.broadcast_to(idx, (_BLK, _NE))


def kernel(x, W, b):
    b2 = b.reshape(1, _NE)
    grid = (_TOKENS // _BLK,)
    router, idx = pl.pallas_call(
        _router_body,
        grid=grid,
        in_specs=[
            pl.BlockSpec((_BLK, _DIM), lambda i: (i, 0)),
            pl.BlockSpec((_DIM, _NE), lambda i: (0, 0)),
            pl.BlockSpec((1, _NE), lambda i: (0, 0)),
        ],
        out_specs=[
            pl.BlockSpec((_BLK, _NE), lambda i: (i, 0)),
            pl.BlockSpec((_BLK, _NE), lambda i: (i, 0)),
        ],
        out_shape=[
            jax.ShapeDtypeStruct((_TOKENS, _NE), jnp.float32),
            jax.ShapeDtypeStruct((_TOKENS, _NE), jnp.int32),
        ],
        compiler_params=pltpu.CompilerParams(
            dimension_semantics=("arbitrary",),
        ),
    )(x, W, b2)
    return (router, idx[:, 0:1])
